# baseline (device time: 37725 ns/iter reference)
import jax
import jax.numpy as jnp
from jax import lax
from jax.experimental import pallas as pl
from jax.experimental.pallas import tpu as pltpu

CHUNKS = (32,) * 15 + (16, 8, 8)
OFFS = tuple(sum(CHUNKS[:i]) for i in range(len(CHUNKS)))
NC = len(CHUNKS)


def kernel(partial, resid, gamma):
    _, m, d = partial.shape
    half = m // 2
    g = gamma.reshape(1, d)

    def body(p_ref, r_ref, g_ref, o_ref, peer_ref,
             a_send, a_recv, b_send, b_recv):
        my_x = lax.axis_index("x")
        my_y = lax.axis_index("y")
        y_nbr = (my_x, 1 - my_y)
        x_nbr = (1 - my_x, my_y)

        barrier_sem = pltpu.get_barrier_semaphore()
        for nbr in (y_nbr, x_nbr):
            pl.semaphore_signal(
                barrier_sem, inc=1, device_id=nbr,
                device_id_type=pl.DeviceIdType.MESH,
            )
        pl.semaphore_wait(barrier_sem, 2)

        base = my_x * half

        a_rdmas = []
        for c in range(NC):
            a = pltpu.make_async_remote_copy(
                src_ref=p_ref.at[0, pl.ds(base + OFFS[c], CHUNKS[c])],
                dst_ref=peer_ref.at[pl.ds(OFFS[c], CHUNKS[c])],
                send_sem=a_send.at[c],
                recv_sem=a_recv.at[c],
                device_id=y_nbr,
                device_id_type=pl.DeviceIdType.MESH,
            )
            a.start()
            a_rdmas.append(a)

        b_rdmas = []
        for c in range(NC):
            a_rdmas[c].wait_recv()
            rows = pl.ds(base + OFFS[c], CHUNKS[c])
            y = (p_ref[0, rows, :] + peer_ref[pl.ds(OFFS[c], CHUNKS[c]), :]
                 + r_ref[rows, :])
            inv = lax.rsqrt(jnp.sum(y * y, axis=-1, keepdims=True) * (1.0 / d)
                            + 1e-6)
            o_ref[rows, :] = y * inv * g_ref[...]
            b = pltpu.make_async_remote_copy(
                src_ref=o_ref.at[rows],
                dst_ref=o_ref.at[rows],
                send_sem=b_send.at[c],
                recv_sem=b_recv.at[c],
                device_id=x_nbr,
                device_id_type=pl.DeviceIdType.MESH,
            )
            b.start()
            b_rdmas.append(b)

        for c in range(NC):
            a_rdmas[c].wait_send()
            b_rdmas[c].wait_send()
            b_rdmas[c].wait_recv()

    return pl.pallas_call(
        body,
        out_shape=jax.ShapeDtypeStruct((m, d), jnp.float32),
        in_specs=[
            pl.BlockSpec(memory_space=pltpu.VMEM),
            pl.BlockSpec(memory_space=pltpu.VMEM),
            pl.BlockSpec(memory_space=pltpu.VMEM),
        ],
        out_specs=pl.BlockSpec(memory_space=pltpu.VMEM),
        scratch_shapes=[
            pltpu.VMEM((half, d), jnp.float32),
            pltpu.SemaphoreType.DMA((NC,)),
            pltpu.SemaphoreType.DMA((NC,)),
            pltpu.SemaphoreType.DMA((NC,)),
            pltpu.SemaphoreType.DMA((NC,)),
        ],
        compiler_params=pltpu.CompilerParams(collective_id=0),
    )(partial, resid, g)
